# baseline (speedup 1.0000x reference)
import functools

import jax
import jax.numpy as jnp
from jax import lax
from jax.experimental import pallas as pl
from jax.experimental.pallas import tpu as pltpu
from jax.experimental.pallas import tpu_sc as plsc

BATCH = 4096
SEQ = 200
EMBED = 64
LANES = 16
NUM_CORES = 2
NUM_SUBCORES = 16
NUM_WORKERS = NUM_CORES * NUM_SUBCORES
SEQ_PER_WORKER = BATCH // NUM_WORKERS

_mesh = plsc.VectorSubcoreMesh(
    core_axis_name="c", subcore_axis_name="s",
    num_cores=NUM_CORES, num_subcores=NUM_SUBCORES,
)


@functools.partial(
    pl.kernel,
    out_type=jax.ShapeDtypeStruct((BATCH, SEQ, EMBED), jnp.float32),
    mesh=_mesh,
    scratch_types=[
        pltpu.VMEM((SEQ, EMBED), jnp.float32),
        pltpu.VMEM((SEQ_PER_WORKER, SEQ), jnp.int32),
        pltpu.VMEM((2, SEQ, EMBED), jnp.float32),
        pltpu.VMEM((2, SEQ, EMBED), jnp.float32),
        pltpu.SemaphoreType.DMA((2,)),
        pltpu.SemaphoreType.DMA((2,)),
    ],
    compiler_params=pltpu.CompilerParams(use_tc_tiling_on_sc=False),
)
def _embed_sc(idx_hbm, table_hbm, pos_hbm, out_hbm,
              pos_v, idx_v, rows_v, ow_v, gsem, osem):
    wid = lax.axis_index("s") * NUM_CORES + lax.axis_index("c")
    base = wid * SEQ_PER_WORKER
    pltpu.sync_copy(pos_hbm, pos_v)
    pltpu.sync_copy(idx_hbm.at[pl.ds(base, SEQ_PER_WORKER)], idx_v)
    pltpu.async_copy(table_hbm.at[idx_v.at[0]], rows_v.at[0], gsem.at[0])

    def seq_body(i, carry):
        p = lax.rem(i, 2)
        q = 1 - p

        @pl.when(i + 1 < SEQ_PER_WORKER)
        def _prefetch():
            pltpu.async_copy(
                table_hbm.at[idx_v.at[i + 1]], rows_v.at[q], gsem.at[q])

        pltpu.make_async_copy(
            table_hbm.at[idx_v.at[i]], rows_v.at[p], gsem.at[p]).wait()

        @pl.when(i >= 2)
        def _drain():
            pltpu.make_async_copy(
                ow_v.at[p], out_hbm.at[base + i - 2], osem.at[p]).wait()

        def row_body(r, c):
            for u in range(2):
                row = 2 * r + u
                for k in range(EMBED // LANES):
                    sl = pl.ds(k * LANES, LANES)
                    ow_v[p, row, sl] = rows_v[p, row, sl] + pos_v[row, sl]
            return c

        lax.fori_loop(0, SEQ // 2, row_body, 0)
        pltpu.async_copy(ow_v.at[p], out_hbm.at[base + i], osem.at[p])
        return carry

    lax.fori_loop(0, SEQ_PER_WORKER, seq_body, 0)
    pltpu.make_async_copy(
        ow_v.at[0], out_hbm.at[base + SEQ_PER_WORKER - 2], osem.at[0]).wait()
    pltpu.make_async_copy(
        ow_v.at[1], out_hbm.at[base + SEQ_PER_WORKER - 1], osem.at[1]).wait()


def kernel(input, word_table, pos_table):
    return _embed_sc(input.astype(jnp.int32), word_table, pos_table)


# speedup vs baseline: 1.3150x; 1.3150x over previous
import functools

import jax
import jax.numpy as jnp
from jax import lax
from jax.experimental import pallas as pl
from jax.experimental.pallas import tpu as pltpu
from jax.experimental.pallas import tpu_sc as plsc

BATCH = 4096
SEQ = 200
EMBED = 64
LANES = 16
NUM_CORES = 2
NUM_SUBCORES = 16
NUM_WORKERS = NUM_CORES * NUM_SUBCORES
ROWS = BATCH * SEQ
ROWS_PER_WORKER = ROWS // NUM_WORKERS
CHUNK = 2 * SEQ
NCHUNKS = ROWS_PER_WORKER // CHUNK
NBUF = 3

_mesh = plsc.VectorSubcoreMesh(
    core_axis_name="c", subcore_axis_name="s",
    num_cores=NUM_CORES, num_subcores=NUM_SUBCORES,
)


@functools.partial(
    pl.kernel,
    out_type=jax.ShapeDtypeStruct((ROWS, EMBED), jnp.float32),
    mesh=_mesh,
    scratch_types=[
        pltpu.VMEM((CHUNK, EMBED), jnp.float32),
        pltpu.VMEM((ROWS_PER_WORKER,), jnp.int32),
        pltpu.VMEM((NBUF, CHUNK, EMBED), jnp.float32),
        pltpu.SemaphoreType.DMA((NBUF,)),
        pltpu.SemaphoreType.DMA((NBUF,)),
    ],
    compiler_params=pltpu.CompilerParams(use_tc_tiling_on_sc=False),
)
def _embed_sc(idx_hbm, table_hbm, pos_hbm, out_hbm,
              pos_v, idx_v, rows_v, gsem, osem):
    wid = lax.axis_index("s") * NUM_CORES + lax.axis_index("c")
    base = wid * ROWS_PER_WORKER
    pltpu.sync_copy(pos_hbm, pos_v.at[pl.ds(0, SEQ)])
    pltpu.sync_copy(pos_hbm, pos_v.at[pl.ds(SEQ, SEQ)])
    pltpu.sync_copy(idx_hbm.at[pl.ds(base, ROWS_PER_WORKER)], idx_v)
    pltpu.async_copy(
        table_hbm.at[idx_v.at[pl.ds(0, CHUNK)]], rows_v.at[0], gsem.at[0])

    def chunk_body(i, carry):
        p = lax.rem(i, NBUF)
        q = lax.rem(i + 1, NBUF)

        @pl.when(i + 1 < NCHUNKS)
        def _prefetch():
            @pl.when(i >= NBUF - 1)
            def _free():
                pltpu.make_async_copy(
                    rows_v.at[q],
                    out_hbm.at[pl.ds(base + (i + 1 - NBUF) * CHUNK, CHUNK)],
                    osem.at[q]).wait()
            pltpu.async_copy(
                table_hbm.at[idx_v.at[pl.ds((i + 1) * CHUNK, CHUNK)]],
                rows_v.at[q], gsem.at[q])

        pltpu.make_async_copy(
            table_hbm.at[idx_v.at[pl.ds(i * CHUNK, CHUNK)]],
            rows_v.at[p], gsem.at[p]).wait()

        @plsc.parallel_loop(0, CHUNK, 1, unroll=4)
        def _add(r):
            for k in range(EMBED // LANES):
                sl = pl.ds(k * LANES, LANES)
                plsc.addupdate(rows_v.at[p, r, sl], pos_v[r, sl])

        pltpu.async_copy(
            rows_v.at[p], out_hbm.at[pl.ds(base + i * CHUNK, CHUNK)],
            osem.at[p])
        return carry

    lax.fori_loop(0, NCHUNKS, chunk_body, 0)
    for j in range(NCHUNKS - NBUF, NCHUNKS):
        pltpu.make_async_copy(
            rows_v.at[j % NBUF],
            out_hbm.at[pl.ds(base + j * CHUNK, CHUNK)],
            osem.at[j % NBUF]).wait()


def kernel(input, word_table, pos_table):
    flat = _embed_sc(input.reshape(-1).astype(jnp.int32),
                     word_table, pos_table)
    return flat.reshape(BATCH, SEQ, EMBED)


# speedup vs baseline: 1.7062x; 1.2975x over previous
import functools

import jax
import jax.numpy as jnp
from jax import lax
from jax.experimental import pallas as pl
from jax.experimental.pallas import tpu as pltpu
from jax.experimental.pallas import tpu_sc as plsc

BATCH = 4096
SEQ = 200
EMBED = 64
VOCAB = 1000000
LANES = 16
NUM_CORES = 2
NUM_SUBCORES = 16
NUM_WORKERS = NUM_CORES * NUM_SUBCORES
ROWS = BATCH * SEQ
ROWS_PER_WORKER = ROWS // NUM_WORKERS
CHUNK = 128
NCHUNKS = ROWS_PER_WORKER // CHUNK
PCH = 2048

_mesh = plsc.VectorSubcoreMesh(
    core_axis_name="c", subcore_axis_name="s",
    num_cores=NUM_CORES, num_subcores=NUM_SUBCORES,
)


@functools.partial(
    pl.kernel,
    out_type=jax.ShapeDtypeStruct((ROWS, EMBED), jnp.float32),
    mesh=_mesh,
    scratch_types=[
        pltpu.VMEM((SEQ, EMBED), jnp.float32),
        pltpu.VMEM((NCHUNKS, CHUNK), jnp.int32),
        pltpu.VMEM((2, CHUNK, 2 * EMBED), jnp.float32),
        pltpu.VMEM((2, CHUNK, EMBED), jnp.float32),
        pltpu.SemaphoreType.DMA((2,)),
        pltpu.SemaphoreType.DMA((2,)),
    ],
    compiler_params=pltpu.CompilerParams(needs_layout_passes=False),
)
def _embed_sc(idx2_hbm, table_hbm, pos_hbm, out_hbm,
              pos_v, idx_v, rows_v, ow_v, gsem, osem):
    wid = lax.axis_index("s") * NUM_CORES + lax.axis_index("c")
    base = wid * ROWS_PER_WORKER
    pltpu.sync_copy(pos_hbm, pos_v)
    pltpu.sync_copy(idx2_hbm.at[pl.ds(wid * NCHUNKS, NCHUNKS)], idx_v)
    pltpu.async_copy(
        table_hbm.at[idx_v.at[0]], rows_v.at[0], gsem.at[0])

    def chunk_body(i, carry):
        p = lax.rem(i, 2)
        q = 1 - p

        @pl.when(i + 1 < NCHUNKS)
        def _prefetch():
            pltpu.async_copy(
                table_hbm.at[idx_v.at[i + 1]], rows_v.at[q], gsem.at[q])

        pltpu.make_async_copy(
            table_hbm.at[idx_v.at[i]], rows_v.at[p], gsem.at[p]).wait()

        @pl.when(i >= 2)
        def _free():
            pltpu.make_async_copy(
                ow_v.at[p],
                out_hbm.at[pl.ds(base + (i - 2) * CHUNK, CHUNK)],
                osem.at[p]).wait()

        off = lax.rem(i * CHUNK, SEQ)
        n1 = lax.min(SEQ - off, CHUNK)

        def add_row(r, srow):
            for k in range(EMBED // LANES):
                sl = pl.ds(k * LANES, LANES)
                ow_v[p, r, sl] = rows_v[p, r, sl] + pos_v[srow, sl]

        @plsc.parallel_loop(0, n1, 1, unroll=2)
        def _seg1(r):
            add_row(r, off + r)

        @plsc.parallel_loop(n1, CHUNK, 1, unroll=2)
        def _seg2(r):
            add_row(r, off + r - SEQ)

        pltpu.async_copy(
            ow_v.at[p], out_hbm.at[pl.ds(base + i * CHUNK, CHUNK)],
            osem.at[p])
        return carry

    lax.fori_loop(0, NCHUNKS, chunk_body, 0)
    for j in (NCHUNKS - 2, NCHUNKS - 1):
        pltpu.make_async_copy(
            ow_v.at[j % 2],
            out_hbm.at[pl.ds(base + j * CHUNK, CHUNK)],
            osem.at[j % 2]).wait()


def _pack_tc_body(t_ref, o_ref):
    x = t_ref[...]
    o_ref[...] = jnp.concatenate(
        [jnp.transpose(x), jnp.zeros((PCH, EMBED), jnp.float32)], axis=1)


def kernel(input, word_table, pos_table):
    table128 = pl.pallas_call(
        _pack_tc_body,
        grid=(-(-VOCAB // PCH),),
        in_specs=[pl.BlockSpec((EMBED, PCH), lambda i: (0, i))],
        out_specs=pl.BlockSpec((PCH, 2 * EMBED), lambda i: (i, 0)),
        out_shape=jax.ShapeDtypeStruct((VOCAB, 2 * EMBED), jnp.float32),
    )(word_table.T)
    idx2 = input.reshape(-1).astype(jnp.int32).reshape(ROWS // CHUNK, CHUNK)
    flat = _embed_sc(idx2, table128, pos_table)
    return flat.reshape(BATCH, SEQ, EMBED)


# speedup vs baseline: 1.9640x; 1.1511x over previous
import functools

import jax
import jax.numpy as jnp
from jax import lax
from jax.experimental import pallas as pl
from jax.experimental.pallas import tpu as pltpu
from jax.experimental.pallas import tpu_sc as plsc

BATCH = 4096
SEQ = 200
EMBED = 64
VOCAB = 1000000
LANES = 16
NUM_CORES = 2
NUM_SUBCORES = 16
NUM_WORKERS = NUM_CORES * NUM_SUBCORES
ROWS = BATCH * SEQ
ROWS_PER_WORKER = ROWS // NUM_WORKERS
CHUNK = 128
NCHUNKS = ROWS_PER_WORKER // CHUNK
PCH = 4096

_mesh = plsc.VectorSubcoreMesh(
    core_axis_name="c", subcore_axis_name="s",
    num_cores=NUM_CORES, num_subcores=NUM_SUBCORES,
)


@functools.partial(
    pl.kernel,
    out_type=jax.ShapeDtypeStruct((ROWS, EMBED), jnp.float32),
    mesh=_mesh,
    scratch_types=[
        pltpu.VMEM((SEQ, EMBED), jnp.float32),
        pltpu.VMEM((NCHUNKS, CHUNK), jnp.int32),
        pltpu.VMEM((2, CHUNK, 2 * EMBED), jnp.float32),
        pltpu.VMEM((2, CHUNK, EMBED), jnp.float32),
        pltpu.SemaphoreType.DMA((2,)),
        pltpu.SemaphoreType.DMA((2,)),
    ],
    compiler_params=pltpu.CompilerParams(needs_layout_passes=False),
)
def _embed_sc(idx2_hbm, table_hbm, pos_hbm, out_hbm,
              pos_v, idx_v, rows_v, ow_v, gsem, osem):
    wid = lax.axis_index("s") * NUM_CORES + lax.axis_index("c")
    base = wid * ROWS_PER_WORKER
    pltpu.sync_copy(pos_hbm, pos_v)
    pltpu.sync_copy(idx2_hbm.at[pl.ds(wid * NCHUNKS, NCHUNKS)], idx_v)
    pltpu.async_copy(
        table_hbm.at[idx_v.at[0]], rows_v.at[0], gsem.at[0])

    def chunk_body(i, carry):
        p = lax.rem(i, 2)
        q = 1 - p

        @pl.when(i + 1 < NCHUNKS)
        def _prefetch():
            pltpu.async_copy(
                table_hbm.at[idx_v.at[i + 1]], rows_v.at[q], gsem.at[q])

        pltpu.make_async_copy(
            table_hbm.at[idx_v.at[i]], rows_v.at[p], gsem.at[p]).wait()

        @pl.when(i >= 2)
        def _free():
            pltpu.make_async_copy(
                ow_v.at[p],
                out_hbm.at[pl.ds(base + (i - 2) * CHUNK, CHUNK)],
                osem.at[p]).wait()

        off = lax.rem(i * CHUNK, SEQ)
        n1 = lax.min(SEQ - off, CHUNK)

        def add_row(r, srow):
            for k in range(EMBED // LANES):
                sl = pl.ds(k * LANES, LANES)
                ow_v[p, r, sl] = rows_v[p, r, sl] + pos_v[srow, sl]

        @plsc.parallel_loop(0, n1, 1, unroll=2)
        def _seg1(r):
            add_row(r, off + r)

        @plsc.parallel_loop(n1, CHUNK, 1, unroll=2)
        def _seg2(r):
            add_row(r, off + r - SEQ)

        pltpu.async_copy(
            ow_v.at[p], out_hbm.at[pl.ds(base + i * CHUNK, CHUNK)],
            osem.at[p])
        return carry

    lax.fori_loop(0, NCHUNKS, chunk_body, 0)
    for j in (NCHUNKS - 2, NCHUNKS - 1):
        pltpu.make_async_copy(
            ow_v.at[j % 2],
            out_hbm.at[pl.ds(base + j * CHUNK, CHUNK)],
            osem.at[j % 2]).wait()


def _pack_tc_body(t_ref, o_ref):
    x = t_ref[...]
    o_ref[...] = jnp.concatenate(
        [jnp.transpose(x), jnp.zeros((PCH, EMBED), jnp.float32)], axis=1)


def kernel(input, word_table, pos_table):
    table128 = pl.pallas_call(
        _pack_tc_body,
        grid=(-(-VOCAB // PCH),),
        in_specs=[pl.BlockSpec((EMBED, PCH), lambda i: (0, i))],
        out_specs=pl.BlockSpec((PCH, 2 * EMBED), lambda i: (i, 0)),
        out_shape=jax.ShapeDtypeStruct((VOCAB, 2 * EMBED), jnp.float32),
    )(word_table.T)
    idx2 = input.reshape(-1).astype(jnp.int32).reshape(ROWS // CHUNK, CHUNK)
    flat = _embed_sc(idx2, table128, pos_table)
    return flat.reshape(BATCH, SEQ, EMBED)


# speedup vs baseline: 2.1533x; 1.0963x over previous
import functools

import jax
import jax.numpy as jnp
from jax import lax
from jax.experimental import pallas as pl
from jax.experimental.pallas import tpu as pltpu
from jax.experimental.pallas import tpu_sc as plsc

BATCH = 4096
SEQ = 200
EMBED = 64
VOCAB = 1000000
LANES = 16
NUM_CORES = 2
NUM_SUBCORES = 16
NUM_WORKERS = NUM_CORES * NUM_SUBCORES
ROWS = BATCH * SEQ
ROWS_PER_WORKER = ROWS // NUM_WORKERS
CHUNK = 128
NCHUNKS = ROWS_PER_WORKER // CHUNK
PCH = 8192

_mesh = plsc.VectorSubcoreMesh(
    core_axis_name="c", subcore_axis_name="s",
    num_cores=NUM_CORES, num_subcores=NUM_SUBCORES,
)


@functools.partial(
    pl.kernel,
    out_type=jax.ShapeDtypeStruct((ROWS, EMBED), jnp.float32),
    mesh=_mesh,
    scratch_types=[
        pltpu.VMEM((SEQ, EMBED), jnp.float32),
        pltpu.VMEM((NCHUNKS, CHUNK), jnp.int32),
        pltpu.VMEM((2, CHUNK, 2 * EMBED), jnp.float32),
        pltpu.VMEM((2, CHUNK, EMBED), jnp.float32),
        pltpu.SemaphoreType.DMA((2,)),
        pltpu.SemaphoreType.DMA((2,)),
    ],
    compiler_params=pltpu.CompilerParams(needs_layout_passes=False),
)
def _embed_sc(idx2_hbm, table_hbm, pos_hbm, out_hbm,
              pos_v, idx_v, rows_v, ow_v, gsem, osem):
    wid = lax.axis_index("s") * NUM_CORES + lax.axis_index("c")
    base = wid * ROWS_PER_WORKER
    pltpu.sync_copy(pos_hbm, pos_v)
    pltpu.sync_copy(idx2_hbm.at[pl.ds(wid * NCHUNKS, NCHUNKS)], idx_v)
    pltpu.async_copy(
        table_hbm.at[idx_v.at[0]], rows_v.at[0], gsem.at[0])

    def chunk_body(i, carry):
        p = lax.rem(i, 2)
        q = 1 - p

        @pl.when(i + 1 < NCHUNKS)
        def _prefetch():
            pltpu.async_copy(
                table_hbm.at[idx_v.at[i + 1]], rows_v.at[q], gsem.at[q])

        pltpu.make_async_copy(
            table_hbm.at[idx_v.at[i]], rows_v.at[p], gsem.at[p]).wait()

        @pl.when(i >= 2)
        def _free():
            pltpu.make_async_copy(
                ow_v.at[p],
                out_hbm.at[pl.ds(base + (i - 2) * CHUNK, CHUNK)],
                osem.at[p]).wait()

        off = lax.rem(i * CHUNK, SEQ)
        n1 = lax.min(SEQ - off, CHUNK)

        def add_row(r, srow):
            for k in range(EMBED // LANES):
                sl = pl.ds(k * LANES, LANES)
                ow_v[p, r, sl] = rows_v[p, r, sl] + pos_v[srow, sl]

        @plsc.parallel_loop(0, n1, 1, unroll=2)
        def _seg1(r):
            add_row(r, off + r)

        @plsc.parallel_loop(n1, CHUNK, 1, unroll=2)
        def _seg2(r):
            add_row(r, off + r - SEQ)

        pltpu.async_copy(
            ow_v.at[p], out_hbm.at[pl.ds(base + i * CHUNK, CHUNK)],
            osem.at[p])
        return carry

    lax.fori_loop(0, NCHUNKS, chunk_body, 0)
    for j in (NCHUNKS - 2, NCHUNKS - 1):
        pltpu.make_async_copy(
            ow_v.at[j % 2],
            out_hbm.at[pl.ds(base + j * CHUNK, CHUNK)],
            osem.at[j % 2]).wait()


def _pack_tc_body(t_ref, o_ref):
    x = t_ref[...]
    o_ref[...] = jnp.concatenate(
        [jnp.transpose(x), jnp.zeros((PCH, EMBED), jnp.float32)], axis=1)


def kernel(input, word_table, pos_table):
    table128 = pl.pallas_call(
        _pack_tc_body,
        grid=(-(-VOCAB // PCH),),
        in_specs=[pl.BlockSpec((EMBED, PCH), lambda i: (0, i))],
        out_specs=pl.BlockSpec((PCH, 2 * EMBED), lambda i: (i, 0)),
        out_shape=jax.ShapeDtypeStruct((VOCAB, 2 * EMBED), jnp.float32),
    )(word_table.T)
    idx2 = input.reshape(-1).astype(jnp.int32).reshape(ROWS // CHUNK, CHUNK)
    flat = _embed_sc(idx2, table128, pos_table)
    return flat.reshape(BATCH, SEQ, EMBED)


# speedup vs baseline: 2.2097x; 1.0262x over previous
import functools

import jax
import jax.numpy as jnp
from jax import lax
from jax.experimental import pallas as pl
from jax.experimental.pallas import tpu as pltpu
from jax.experimental.pallas import tpu_sc as plsc

BATCH = 4096
SEQ = 200
EMBED = 64
VOCAB = 1000000
LANES = 16
NUM_CORES = 2
NUM_SUBCORES = 16
NUM_WORKERS = NUM_CORES * NUM_SUBCORES
ROWS = BATCH * SEQ
ROWS_PER_WORKER = ROWS // NUM_WORKERS
CHUNK = 128
NCHUNKS = ROWS_PER_WORKER // CHUNK
PCH = 16384

_mesh = plsc.VectorSubcoreMesh(
    core_axis_name="c", subcore_axis_name="s",
    num_cores=NUM_CORES, num_subcores=NUM_SUBCORES,
)


@functools.partial(
    pl.kernel,
    out_type=jax.ShapeDtypeStruct((ROWS, EMBED), jnp.float32),
    mesh=_mesh,
    scratch_types=[
        pltpu.VMEM((SEQ, EMBED), jnp.float32),
        pltpu.VMEM((NCHUNKS, CHUNK), jnp.int32),
        pltpu.VMEM((2, CHUNK, 2 * EMBED), jnp.float32),
        pltpu.VMEM((2, CHUNK, EMBED), jnp.float32),
        pltpu.SemaphoreType.DMA((2,)),
        pltpu.SemaphoreType.DMA((2,)),
    ],
    compiler_params=pltpu.CompilerParams(needs_layout_passes=False),
)
def _embed_sc(idx2_hbm, table_hbm, pos_hbm, out_hbm,
              pos_v, idx_v, rows_v, ow_v, gsem, osem):
    wid = lax.axis_index("s") * NUM_CORES + lax.axis_index("c")
    base = wid * ROWS_PER_WORKER
    pltpu.sync_copy(pos_hbm, pos_v)
    pltpu.sync_copy(idx2_hbm.at[pl.ds(wid * NCHUNKS, NCHUNKS)], idx_v)
    pltpu.async_copy(
        table_hbm.at[idx_v.at[0]], rows_v.at[0], gsem.at[0])

    def chunk_body(i, carry):
        p = lax.rem(i, 2)
        q = 1 - p

        @pl.when(i + 1 < NCHUNKS)
        def _prefetch():
            pltpu.async_copy(
                table_hbm.at[idx_v.at[i + 1]], rows_v.at[q], gsem.at[q])

        pltpu.make_async_copy(
            table_hbm.at[idx_v.at[i]], rows_v.at[p], gsem.at[p]).wait()

        @pl.when(i >= 2)
        def _free():
            pltpu.make_async_copy(
                ow_v.at[p],
                out_hbm.at[pl.ds(base + (i - 2) * CHUNK, CHUNK)],
                osem.at[p]).wait()

        off = lax.rem(i * CHUNK, SEQ)
        n1 = lax.min(SEQ - off, CHUNK)

        def add_row(r, srow):
            for k in range(EMBED // LANES):
                sl = pl.ds(k * LANES, LANES)
                ow_v[p, r, sl] = rows_v[p, r, sl] + pos_v[srow, sl]

        @plsc.parallel_loop(0, n1, 1, unroll=2)
        def _seg1(r):
            add_row(r, off + r)

        @plsc.parallel_loop(n1, CHUNK, 1, unroll=2)
        def _seg2(r):
            add_row(r, off + r - SEQ)

        pltpu.async_copy(
            ow_v.at[p], out_hbm.at[pl.ds(base + i * CHUNK, CHUNK)],
            osem.at[p])
        return carry

    lax.fori_loop(0, NCHUNKS, chunk_body, 0)
    for j in (NCHUNKS - 2, NCHUNKS - 1):
        pltpu.make_async_copy(
            ow_v.at[j % 2],
            out_hbm.at[pl.ds(base + j * CHUNK, CHUNK)],
            osem.at[j % 2]).wait()


def _pack_tc_body(t_ref, o_ref):
    x = t_ref[...]
    o_ref[...] = jnp.concatenate(
        [jnp.transpose(x), jnp.zeros((PCH, EMBED), jnp.float32)], axis=1)


def kernel(input, word_table, pos_table):
    table128 = pl.pallas_call(
        _pack_tc_body,
        grid=(-(-VOCAB // PCH),),
        in_specs=[pl.BlockSpec((EMBED, PCH), lambda i: (0, i))],
        out_specs=pl.BlockSpec((PCH, 2 * EMBED), lambda i: (i, 0)),
        out_shape=jax.ShapeDtypeStruct((VOCAB, 2 * EMBED), jnp.float32),
    )(word_table.T)
    idx2 = input.reshape(-1).astype(jnp.int32).reshape(ROWS // CHUNK, CHUNK)
    flat = _embed_sc(idx2, table128, pos_table)
    return flat.reshape(BATCH, SEQ, EMBED)


# speedup vs baseline: 2.2257x; 1.0072x over previous
import functools

import jax
import jax.numpy as jnp
from jax import lax
from jax.experimental import pallas as pl
from jax.experimental.pallas import tpu as pltpu
from jax.experimental.pallas import tpu_sc as plsc

BATCH = 4096
SEQ = 200
EMBED = 64
VOCAB = 1000000
LANES = 16
NUM_CORES = 2
NUM_SUBCORES = 16
NUM_WORKERS = NUM_CORES * NUM_SUBCORES
ROWS = BATCH * SEQ
ROWS_PER_WORKER = ROWS // NUM_WORKERS
CHUNK = 128
NCHUNKS = ROWS_PER_WORKER // CHUNK
PCH = 32768

_mesh = plsc.VectorSubcoreMesh(
    core_axis_name="c", subcore_axis_name="s",
    num_cores=NUM_CORES, num_subcores=NUM_SUBCORES,
)


@functools.partial(
    pl.kernel,
    out_type=jax.ShapeDtypeStruct((ROWS, EMBED), jnp.float32),
    mesh=_mesh,
    scratch_types=[
        pltpu.VMEM((SEQ, EMBED), jnp.float32),
        pltpu.VMEM((NCHUNKS, CHUNK), jnp.int32),
        pltpu.VMEM((2, CHUNK, 2 * EMBED), jnp.float32),
        pltpu.VMEM((2, CHUNK, EMBED), jnp.float32),
        pltpu.SemaphoreType.DMA((2,)),
        pltpu.SemaphoreType.DMA((2,)),
    ],
    compiler_params=pltpu.CompilerParams(needs_layout_passes=False),
)
def _embed_sc(idx2_hbm, table_hbm, pos_hbm, out_hbm,
              pos_v, idx_v, rows_v, ow_v, gsem, osem):
    wid = lax.axis_index("s") * NUM_CORES + lax.axis_index("c")
    base = wid * ROWS_PER_WORKER
    pltpu.sync_copy(pos_hbm, pos_v)
    pltpu.sync_copy(idx2_hbm.at[pl.ds(wid * NCHUNKS, NCHUNKS)], idx_v)
    pltpu.async_copy(
        table_hbm.at[idx_v.at[0]], rows_v.at[0], gsem.at[0])

    def chunk_body(i, carry):
        p = lax.rem(i, 2)
        q = 1 - p

        @pl.when(i + 1 < NCHUNKS)
        def _prefetch():
            pltpu.async_copy(
                table_hbm.at[idx_v.at[i + 1]], rows_v.at[q], gsem.at[q])

        pltpu.make_async_copy(
            table_hbm.at[idx_v.at[i]], rows_v.at[p], gsem.at[p]).wait()

        @pl.when(i >= 2)
        def _free():
            pltpu.make_async_copy(
                ow_v.at[p],
                out_hbm.at[pl.ds(base + (i - 2) * CHUNK, CHUNK)],
                osem.at[p]).wait()

        off = lax.rem(i * CHUNK, SEQ)
        n1 = lax.min(SEQ - off, CHUNK)

        def add_row(r, srow):
            for k in range(EMBED // LANES):
                sl = pl.ds(k * LANES, LANES)
                ow_v[p, r, sl] = rows_v[p, r, sl] + pos_v[srow, sl]

        @plsc.parallel_loop(0, n1, 1, unroll=2)
        def _seg1(r):
            add_row(r, off + r)

        @plsc.parallel_loop(n1, CHUNK, 1, unroll=2)
        def _seg2(r):
            add_row(r, off + r - SEQ)

        pltpu.async_copy(
            ow_v.at[p], out_hbm.at[pl.ds(base + i * CHUNK, CHUNK)],
            osem.at[p])
        return carry

    lax.fori_loop(0, NCHUNKS, chunk_body, 0)
    for j in (NCHUNKS - 2, NCHUNKS - 1):
        pltpu.make_async_copy(
            ow_v.at[j % 2],
            out_hbm.at[pl.ds(base + j * CHUNK, CHUNK)],
            osem.at[j % 2]).wait()


def _pack_tc_body(t_ref, o_ref):
    x = t_ref[...]
    o_ref[...] = jnp.concatenate(
        [jnp.transpose(x), jnp.zeros((PCH, EMBED), jnp.float32)], axis=1)


def kernel(input, word_table, pos_table):
    table128 = pl.pallas_call(
        _pack_tc_body,
        grid=(-(-VOCAB // PCH),),
        in_specs=[pl.BlockSpec((EMBED, PCH), lambda i: (0, i))],
        out_specs=pl.BlockSpec((PCH, 2 * EMBED), lambda i: (i, 0)),
        out_shape=jax.ShapeDtypeStruct((VOCAB, 2 * EMBED), jnp.float32),
    )(word_table.T)
    idx2 = input.reshape(-1).astype(jnp.int32).reshape(ROWS // CHUNK, CHUNK)
    flat = _embed_sc(idx2, table128, pos_table)
    return flat.reshape(BATCH, SEQ, EMBED)
